# baseline (speedup 1.0000x reference)
import functools

import jax
import jax.numpy as jnp
from jax import lax
from jax.experimental import pallas as pl
from jax.experimental.pallas import tpu as pltpu
from jax.experimental.pallas import tpu_sc as plsc

H = 2048
FF = 4096
E = 8
TOPK = 2
T = 4096
A = T * TOPK
BLK = 128
G = A + E * BLK
NBLK = G // BLK
NBPAD = 80
FFT = 1024
NF = FF // FFT

NC = 2
NS = 16
NW = NC * NS
CHUNK = A // NW


def _router_body(x_ref, gw_ref, logits_ref, topi_ref, topv_ref):
    x = x_ref[...]
    logits = lax.dot_general(x, gw_ref[...], (((1,), (1,)), ((), ())),
                             preferred_element_type=jnp.float32)
    logits_ref[...] = logits
    bt = logits.shape[0]
    idx = lax.broadcasted_iota(jnp.int32, (bt, E), 1)
    m1 = jnp.max(logits, axis=1, keepdims=True)
    i1 = jnp.min(jnp.where(logits == m1, idx, E), axis=1, keepdims=True)
    l2 = jnp.where(idx == i1, -jnp.inf, logits)
    m2 = jnp.max(l2, axis=1, keepdims=True)
    i2 = jnp.min(jnp.where(l2 == m2, idx, E), axis=1, keepdims=True)
    s = 1.0 / (1.0 + jnp.exp(m2 - m1))
    topi_ref[...] = jnp.concatenate([i1, i2], axis=1)
    topv_ref[...] = jnp.concatenate([s, 1.0 - s], axis=1)


def _router(x, gate_w):
    bt = T // 4
    return pl.pallas_call(
        _router_body,
        grid=(4,),
        in_specs=[
            pl.BlockSpec((bt, H), lambda i: (i, 0)),
            pl.BlockSpec((E, H), lambda i: (0, 0)),
        ],
        out_specs=[
            pl.BlockSpec((bt, E), lambda i: (i, 0)),
            pl.BlockSpec((bt, TOPK), lambda i: (i, 0)),
            pl.BlockSpec((bt, TOPK), lambda i: (i, 0)),
        ],
        out_shape=[
            jax.ShapeDtypeStruct((T, E), jnp.float32),
            jax.ShapeDtypeStruct((T, TOPK), jnp.int32),
            jax.ShapeDtypeStruct((T, TOPK), jnp.float32),
        ],
        name="moe_router",
    )(x, gate_w)


def _dispatch_body(topi_hbm, x_hbm, xg_hbm, pos_hbm, bexp_hbm,
                   ids_v, pos_v, rows_v, bexp_v, *sems):
    wid = lax.axis_index("s") * NC + lax.axis_index("c")
    i32 = jnp.int32
    one16 = jnp.ones((16,), i32)
    zero16 = jnp.zeros((16,), i32)
    pltpu.sync_copy(topi_hbm, ids_v)

    def count_body(i, accs):
        v = ids_v[pl.ds(i * 16, 16)]
        return tuple(accs[e] + jnp.where(v == e, one16, zero16)
                     for e in range(E))

    zeros8 = tuple(jnp.zeros((16,), i32) for _ in range(E))
    acc_tot = lax.fori_loop(0, A // 16, count_body, zeros8)
    acc_pre = lax.fori_loop(0, wid * (CHUNK // 16), count_body, zeros8)
    counts = [jnp.sum(acc_tot[e]) for e in range(E)]
    prefix = [jnp.sum(acc_pre[e]) for e in range(E)]

    offs = []
    ends = []
    off = jnp.int32(0)
    for e in range(E):
        offs.append(off)
        off = off + ((counts[e] + (BLK - 1)) // BLK) * BLK
        ends.append(off)

    iota16 = lax.iota(i32, 16)
    base = [offs[e] + prefix[e] for e in range(E)]
    nch = CHUNK // 16
    gh = {}
    sh = {}
    tok0 = (wid * CHUNK + iota16) // TOPK
    gh[0] = pltpu.async_copy(x_hbm.at[tok0], rows_v.at[0], sems[0])
    for v in range(nch):
        buf = v % 2
        j0 = wid * CHUNK + v * 16
        if v + 1 < nch:
            if v >= 1:
                sh[v - 1].wait()
            tokn = (j0 + 16 + iota16) // TOPK
            gh[v + 1] = pltpu.async_copy(
                x_hbm.at[tokn], rows_v.at[1 - buf], sems[1 - buf])
        ids16 = ids_v[pl.ds(j0, 16)]
        pos16 = jnp.zeros((16,), i32)
        for e in range(E):
            m = ids16 == e
            mi = jnp.where(m, one16, zero16)
            incl = plsc.cumsum(mi)
            pos16 = jnp.where(m, base[e] + (incl - mi), pos16)
            base[e] = base[e] + jnp.sum(mi)
        pos_v[pl.ds(v * 16, 16)] = pos16
        gh[v].wait()
        sh[v] = pltpu.async_copy(rows_v.at[buf], xg_hbm.at[pos16], sems[2 + buf])
    sh[nch - 2].wait()
    sh[nch - 1].wait()

    pltpu.sync_copy(pos_v, pos_hbm.at[pl.ds(pl.multiple_of(wid * CHUNK, 8), CHUNK)])

    @pl.when(wid == 0)
    def _():
        for i in range(NBPAD // 16):
            rowstart = (iota16 + i * 16) * BLK
            acc = jnp.zeros((16,), i32)
            for e in range(E):
                acc = acc + jnp.where(rowstart >= ends[e], one16, zero16)
            bexp_v[pl.ds(i * 16, 16)] = acc
        pltpu.sync_copy(bexp_v, bexp_hbm)


def _dispatch(topi_flat, x):
    mesh = plsc.VectorSubcoreMesh(core_axis_name="c", subcore_axis_name="s")
    return pl.kernel(
        _dispatch_body,
        out_type=(
            jax.ShapeDtypeStruct((G, H), jnp.float32),
            jax.ShapeDtypeStruct((A,), jnp.int32),
            jax.ShapeDtypeStruct((NBPAD,), jnp.int32),
        ),
        mesh=mesh,
        scratch_types=[
            pltpu.VMEM((A,), jnp.int32),
            pltpu.VMEM((CHUNK,), jnp.int32),
            pltpu.VMEM((2, 16, H), jnp.float32),
            pltpu.VMEM((NBPAD,), jnp.int32),
            pltpu.SemaphoreType.DMA,
            pltpu.SemaphoreType.DMA,
            pltpu.SemaphoreType.DMA,
            pltpu.SemaphoreType.DMA,
        ],
        compiler_params=pltpu.CompilerParams(needs_layout_passes=False),
        name="moe_dispatch",
    )(topi_flat, x)


def _mlp_init_body(be_ref, xg_ref, w1_ref, w3_ref, w2_ref, yg_ref):
    b = pl.program_id(0)
    e = be_ref[b]

    @pl.when(e < E)
    def _():
        x = xg_ref[...]
        a = lax.dot_general(x, w1_ref[0], (((1,), (1,)), ((), ())),
                            preferred_element_type=jnp.float32)
        c = lax.dot_general(x, w3_ref[0], (((1,), (1,)), ((), ())),
                            preferred_element_type=jnp.float32)
        h = (a * jax.nn.sigmoid(a)) * c
        yg_ref[...] = lax.dot_general(h, w2_ref[0], (((1,), (1,)), ((), ())),
                                      preferred_element_type=jnp.float32)


def _mlp_acc_body(be_ref, xg_ref, w1_ref, w3_ref, w2_ref, yin_ref, yg_ref):
    b = pl.program_id(0)
    e = be_ref[b]

    @pl.when(e < E)
    def _():
        x = xg_ref[...]
        a = lax.dot_general(x, w1_ref[0], (((1,), (1,)), ((), ())),
                            preferred_element_type=jnp.float32)
        c = lax.dot_general(x, w3_ref[0], (((1,), (1,)), ((), ())),
                            preferred_element_type=jnp.float32)
        h = (a * jax.nn.sigmoid(a)) * c
        yg_ref[...] = yin_ref[...] + lax.dot_general(
            h, w2_ref[0], (((1,), (1,)), ((), ())),
            preferred_element_type=jnp.float32)


def _mlp(bexp, xg, w1, w3, w2):
    yg = None
    for f in range(NF):
        base_specs = [
            pl.BlockSpec((BLK, H),
                         lambda b, be: (jnp.where(be[b] < E, b, 0), 0)),
            pl.BlockSpec((1, FFT, H),
                         lambda b, be, f=f: (jnp.minimum(be[b], E - 1), f, 0)),
            pl.BlockSpec((1, FFT, H),
                         lambda b, be, f=f: (jnp.minimum(be[b], E - 1), f, 0)),
            pl.BlockSpec((1, H, FFT),
                         lambda b, be, f=f: (jnp.minimum(be[b], E - 1), 0, f)),
        ]
        out_spec = pl.BlockSpec((BLK, H), lambda b, be: (b, 0))
        if yg is None:
            grid_spec = pltpu.PrefetchScalarGridSpec(
                num_scalar_prefetch=1, grid=(NBLK,),
                in_specs=base_specs, out_specs=out_spec)
            yg = pl.pallas_call(
                _mlp_init_body,
                grid_spec=grid_spec,
                out_shape=jax.ShapeDtypeStruct((G, H), jnp.float32),
                compiler_params=pltpu.CompilerParams(
                    dimension_semantics=("arbitrary",),
                    vmem_limit_bytes=63 * 1024 * 1024),
                name="moe_mlp_f0",
            )(bexp, xg, w1, w3, w2)
        else:
            grid_spec = pltpu.PrefetchScalarGridSpec(
                num_scalar_prefetch=1, grid=(NBLK,),
                in_specs=base_specs + [out_spec], out_specs=out_spec)
            yg = pl.pallas_call(
                _mlp_acc_body,
                grid_spec=grid_spec,
                out_shape=jax.ShapeDtypeStruct((G, H), jnp.float32),
                input_output_aliases={5: 0},
                compiler_params=pltpu.CompilerParams(
                    dimension_semantics=("arbitrary",),
                    vmem_limit_bytes=63 * 1024 * 1024),
                name=f"moe_mlp_f{f}",
            )(bexp, xg, w1, w3, w2, yg)
    return yg


def _combine_body(yg_hbm, pos_hbm, topv_hbm, out_hbm,
                  posc, tvc, rows_v, out_v, *sems):
    wid = lax.axis_index("s") * NC + lax.axis_index("c")
    nch = CHUNK // 16
    base8 = pl.multiple_of(wid * CHUNK, 8)
    pltpu.sync_copy(pos_hbm.at[pl.ds(base8, CHUNK)], posc)
    pltpu.sync_copy(topv_hbm.at[pl.ds(base8, CHUNK)], tvc)
    gh = {}
    so = {}
    gh[0] = pltpu.async_copy(yg_hbm.at[posc[pl.ds(0, 16)]], rows_v.at[0],
                             sems[0])
    for c in range(nch):
        buf = c % 2
        if c + 1 < nch:
            pvn = posc[pl.ds((c + 1) * 16, 16)]
            gh[c + 1] = pltpu.async_copy(yg_hbm.at[pvn], rows_v.at[1 - buf],
                                         sems[1 - buf])
        if c >= 2:
            so[c - 2].wait()
        gh[c].wait()
        tv = tvc[pl.ds(c * 16, 16)]
        svs = [tv[k] for k in range(16)]

        def blend(j, _, buf=buf, svs=svs):
            for i in range(8):
                a = rows_v[buf, 2 * i, pl.ds(j * 16, 16)]
                b = rows_v[buf, 2 * i + 1, pl.ds(j * 16, 16)]
                out_v[buf, i, pl.ds(j * 16, 16)] = (
                    svs[2 * i] * a + svs[2 * i + 1] * b)
            return 0

        lax.fori_loop(0, H // 16, blend, 0)
        so[c] = pltpu.async_copy(
            out_v.at[buf],
            out_hbm.at[pl.ds(pl.multiple_of(base8 // TOPK + c * 8, 8), 8)],
            sems[2 + buf])
    so[nch - 2].wait()
    so[nch - 1].wait()


def _combine(ygflat, pos, topv_flat):
    mesh = plsc.VectorSubcoreMesh(core_axis_name="c", subcore_axis_name="s")
    return pl.kernel(
        _combine_body,
        out_type=jax.ShapeDtypeStruct((T, H), jnp.float32),
        mesh=mesh,
        scratch_types=[
            pltpu.VMEM((CHUNK,), jnp.int32),
            pltpu.VMEM((CHUNK,), jnp.float32),
            pltpu.VMEM((2, 16, H), jnp.float32),
            pltpu.VMEM((2, 8, H), jnp.float32),
            pltpu.SemaphoreType.DMA,
            pltpu.SemaphoreType.DMA,
            pltpu.SemaphoreType.DMA,
            pltpu.SemaphoreType.DMA,
        ],
        compiler_params=pltpu.CompilerParams(needs_layout_passes=False),
        name="moe_combine",
    )(ygflat, pos, topv_flat)


def kernel(hidden_states, gate_w, w1, w2, w3):
    B, S, Hd = hidden_states.shape
    x = hidden_states.reshape(-1, Hd)
    logits, topi, topv = _router(x, gate_w)
    xg, pos, bexp = _dispatch(topi.reshape(-1), x)
    yg = _mlp(bexp, xg, w1, w3, w2)
    out = _combine(yg, pos, topv.reshape(-1))
    return out.reshape(B, S, Hd), logits


# speedup vs baseline: 1.6922x; 1.6922x over previous
import functools

import jax
import jax.numpy as jnp
from jax import lax
from jax.experimental import pallas as pl
from jax.experimental.pallas import tpu as pltpu
from jax.experimental.pallas import tpu_sc as plsc

H = 2048
FF = 4096
E = 8
TOPK = 2
T = 4096
A = T * TOPK
BLK = 256
G = A + E * BLK
NBLK = G // BLK
NBPAD = 48
FFT = 1024
NF = FF // FFT

NC = 2
NS = 16
NW = NC * NS
CHUNK = A // NW


def _router_body(x_ref, gw_ref, logits_ref, topi_ref, topv_ref):
    x = x_ref[...]
    logits = lax.dot_general(x, gw_ref[...], (((1,), (1,)), ((), ())),
                             preferred_element_type=jnp.float32)
    logits_ref[...] = logits
    bt = logits.shape[0]
    idx = lax.broadcasted_iota(jnp.int32, (bt, E), 1)
    m1 = jnp.max(logits, axis=1, keepdims=True)
    i1 = jnp.min(jnp.where(logits == m1, idx, E), axis=1, keepdims=True)
    l2 = jnp.where(idx == i1, -jnp.inf, logits)
    m2 = jnp.max(l2, axis=1, keepdims=True)
    i2 = jnp.min(jnp.where(l2 == m2, idx, E), axis=1, keepdims=True)
    s = 1.0 / (1.0 + jnp.exp(m2 - m1))
    topi_ref[...] = jnp.concatenate([i1, i2], axis=1)
    topv_ref[...] = jnp.concatenate([s, 1.0 - s], axis=1)


def _router(x, gate_w):
    bt = T // 4
    return pl.pallas_call(
        _router_body,
        grid=(4,),
        in_specs=[
            pl.BlockSpec((bt, H), lambda i: (i, 0)),
            pl.BlockSpec((E, H), lambda i: (0, 0)),
        ],
        out_specs=[
            pl.BlockSpec((bt, E), lambda i: (i, 0)),
            pl.BlockSpec((bt, TOPK), lambda i: (i, 0)),
            pl.BlockSpec((bt, TOPK), lambda i: (i, 0)),
        ],
        out_shape=[
            jax.ShapeDtypeStruct((T, E), jnp.float32),
            jax.ShapeDtypeStruct((T, TOPK), jnp.int32),
            jax.ShapeDtypeStruct((T, TOPK), jnp.float32),
        ],
        name="moe_router",
    )(x, gate_w)


def _dispatch_body(topi_hbm, x_hbm, xg_hbm, pos_hbm, bexp_hbm,
                   ids_v, pos_v, rows_v, bexp_v, *sems):
    wid = lax.axis_index("s") * NC + lax.axis_index("c")
    i32 = jnp.int32
    one16 = jnp.ones((16,), i32)
    zero16 = jnp.zeros((16,), i32)
    pltpu.sync_copy(topi_hbm, ids_v)

    def count_body(i, accs):
        v = ids_v[pl.ds(i * 16, 16)]
        return tuple(accs[e] + jnp.where(v == e, one16, zero16)
                     for e in range(E))

    zeros8 = tuple(jnp.zeros((16,), i32) for _ in range(E))
    acc_tot = lax.fori_loop(0, A // 16, count_body, zeros8)
    acc_pre = lax.fori_loop(0, wid * (CHUNK // 16), count_body, zeros8)
    counts = [jnp.sum(acc_tot[e]) for e in range(E)]
    prefix = [jnp.sum(acc_pre[e]) for e in range(E)]

    offs = []
    ends = []
    off = jnp.int32(0)
    for e in range(E):
        offs.append(off)
        off = off + ((counts[e] + (BLK - 1)) // BLK) * BLK
        ends.append(off)

    iota16 = lax.iota(i32, 16)
    base = [offs[e] + prefix[e] for e in range(E)]
    nch = CHUNK // 16
    gh = {}
    sh = {}
    tok0 = (wid * CHUNK + iota16) // TOPK
    gh[0] = pltpu.async_copy(x_hbm.at[tok0], rows_v.at[0], sems[0])
    for v in range(nch):
        buf = v % 2
        j0 = wid * CHUNK + v * 16
        if v + 1 < nch:
            if v >= 1:
                sh[v - 1].wait()
            tokn = (j0 + 16 + iota16) // TOPK
            gh[v + 1] = pltpu.async_copy(
                x_hbm.at[tokn], rows_v.at[1 - buf], sems[1 - buf])
        ids16 = ids_v[pl.ds(j0, 16)]
        pos16 = jnp.zeros((16,), i32)
        for e in range(E):
            m = ids16 == e
            mi = jnp.where(m, one16, zero16)
            incl = plsc.cumsum(mi)
            pos16 = jnp.where(m, base[e] + (incl - mi), pos16)
            base[e] = base[e] + jnp.sum(mi)
        pos_v[pl.ds(v * 16, 16)] = pos16
        gh[v].wait()
        sh[v] = pltpu.async_copy(rows_v.at[buf], xg_hbm.at[pos16], sems[2 + buf])
    sh[nch - 2].wait()
    sh[nch - 1].wait()

    pltpu.sync_copy(pos_v, pos_hbm.at[pl.ds(pl.multiple_of(wid * CHUNK, 8), CHUNK)])

    @pl.when(wid == 0)
    def _():
        for i in range(NBPAD // 16):
            rowstart = (iota16 + i * 16) * BLK
            acc = jnp.zeros((16,), i32)
            for e in range(E):
                acc = acc + jnp.where(rowstart >= ends[e], one16, zero16)
            bexp_v[pl.ds(i * 16, 16)] = acc
        pltpu.sync_copy(bexp_v, bexp_hbm)


def _dispatch(topi_flat, x):
    mesh = plsc.VectorSubcoreMesh(core_axis_name="c", subcore_axis_name="s")
    return pl.kernel(
        _dispatch_body,
        out_type=(
            jax.ShapeDtypeStruct((G, H), jnp.float32),
            jax.ShapeDtypeStruct((A,), jnp.int32),
            jax.ShapeDtypeStruct((NBPAD,), jnp.int32),
        ),
        mesh=mesh,
        scratch_types=[
            pltpu.VMEM((A,), jnp.int32),
            pltpu.VMEM((CHUNK,), jnp.int32),
            pltpu.VMEM((2, 16, H), jnp.float32),
            pltpu.VMEM((NBPAD,), jnp.int32),
            pltpu.SemaphoreType.DMA,
            pltpu.SemaphoreType.DMA,
            pltpu.SemaphoreType.DMA,
            pltpu.SemaphoreType.DMA,
        ],
        compiler_params=pltpu.CompilerParams(needs_layout_passes=False),
        name="moe_dispatch",
    )(topi_flat, x)


def _mlp_init_body(be_ref, xg_ref, w1_ref, w3_ref, w2_ref, yg_ref):
    b = pl.program_id(0)
    e = be_ref[b]

    @pl.when(e < E)
    def _():
        x = xg_ref[...]
        a = lax.dot_general(x, w1_ref[0], (((1,), (1,)), ((), ())),
                            preferred_element_type=jnp.float32)
        c = lax.dot_general(x, w3_ref[0], (((1,), (1,)), ((), ())),
                            preferred_element_type=jnp.float32)
        h = (a * jax.nn.sigmoid(a)) * c
        yg_ref[...] = lax.dot_general(h, w2_ref[0], (((1,), (1,)), ((), ())),
                                      preferred_element_type=jnp.float32)


def _mlp_acc_body(be_ref, xg_ref, w1_ref, w3_ref, w2_ref, yin_ref, yg_ref):
    b = pl.program_id(0)
    e = be_ref[b]

    @pl.when(e < E)
    def _():
        x = xg_ref[...]
        a = lax.dot_general(x, w1_ref[0], (((1,), (1,)), ((), ())),
                            preferred_element_type=jnp.float32)
        c = lax.dot_general(x, w3_ref[0], (((1,), (1,)), ((), ())),
                            preferred_element_type=jnp.float32)
        h = (a * jax.nn.sigmoid(a)) * c
        yg_ref[...] = yin_ref[...] + lax.dot_general(
            h, w2_ref[0], (((1,), (1,)), ((), ())),
            preferred_element_type=jnp.float32)


def _mlp(bexp, xg, w1, w3, w2):
    yg = None
    for f in range(NF):
        base_specs = [
            pl.BlockSpec((BLK, H),
                         lambda b, be: (jnp.where(be[b] < E, b, 0), 0)),
            pl.BlockSpec((1, FFT, H),
                         lambda b, be, f=f: (jnp.minimum(be[b], E - 1), f, 0)),
            pl.BlockSpec((1, FFT, H),
                         lambda b, be, f=f: (jnp.minimum(be[b], E - 1), f, 0)),
            pl.BlockSpec((1, H, FFT),
                         lambda b, be, f=f: (jnp.minimum(be[b], E - 1), 0, f)),
        ]
        out_spec = pl.BlockSpec((BLK, H), lambda b, be: (b, 0))
        if yg is None:
            grid_spec = pltpu.PrefetchScalarGridSpec(
                num_scalar_prefetch=1, grid=(NBLK,),
                in_specs=base_specs, out_specs=out_spec)
            yg = pl.pallas_call(
                _mlp_init_body,
                grid_spec=grid_spec,
                out_shape=jax.ShapeDtypeStruct((G, H), jnp.float32),
                compiler_params=pltpu.CompilerParams(
                    dimension_semantics=("arbitrary",),
                    vmem_limit_bytes=63 * 1024 * 1024),
                name="moe_mlp_f0",
            )(bexp, xg, w1, w3, w2)
        else:
            grid_spec = pltpu.PrefetchScalarGridSpec(
                num_scalar_prefetch=1, grid=(NBLK,),
                in_specs=base_specs + [out_spec], out_specs=out_spec)
            yg = pl.pallas_call(
                _mlp_acc_body,
                grid_spec=grid_spec,
                out_shape=jax.ShapeDtypeStruct((G, H), jnp.float32),
                input_output_aliases={5: 0},
                compiler_params=pltpu.CompilerParams(
                    dimension_semantics=("arbitrary",),
                    vmem_limit_bytes=63 * 1024 * 1024),
                name=f"moe_mlp_f{f}",
            )(bexp, xg, w1, w3, w2, yg)
    return yg


def _combine_body(yg_hbm, pos_hbm, topv_hbm, out_hbm,
                  posc, tvc, rows_v, out_v, *sems):
    wid = lax.axis_index("s") * NC + lax.axis_index("c")
    nch = CHUNK // 16
    base8 = pl.multiple_of(wid * CHUNK, 8)
    pltpu.sync_copy(pos_hbm.at[pl.ds(base8, CHUNK)], posc)
    pltpu.sync_copy(topv_hbm.at[pl.ds(base8, CHUNK)], tvc)
    gh = {}
    so = {}
    gh[0] = pltpu.async_copy(yg_hbm.at[posc[pl.ds(0, 16)]], rows_v.at[0],
                             sems[0])
    for c in range(nch):
        buf = c % 2
        if c + 1 < nch:
            pvn = posc[pl.ds((c + 1) * 16, 16)]
            gh[c + 1] = pltpu.async_copy(yg_hbm.at[pvn], rows_v.at[1 - buf],
                                         sems[1 - buf])
        if c >= 2:
            so[c - 2].wait()
        gh[c].wait()
        tv = tvc[pl.ds(c * 16, 16)]
        svs = [tv[k] for k in range(16)]

        def blend(j, _, buf=buf, svs=svs):
            for i in range(8):
                a = rows_v[buf, 2 * i, pl.ds(j * 16, 16)]
                b = rows_v[buf, 2 * i + 1, pl.ds(j * 16, 16)]
                out_v[buf, i, pl.ds(j * 16, 16)] = (
                    svs[2 * i] * a + svs[2 * i + 1] * b)
            return 0

        lax.fori_loop(0, H // 16, blend, 0)
        so[c] = pltpu.async_copy(
            out_v.at[buf],
            out_hbm.at[pl.ds(pl.multiple_of(base8 // TOPK + c * 8, 8), 8)],
            sems[2 + buf])
    so[nch - 2].wait()
    so[nch - 1].wait()


def _combine(ygflat, pos, topv_flat):
    mesh = plsc.VectorSubcoreMesh(core_axis_name="c", subcore_axis_name="s")
    return pl.kernel(
        _combine_body,
        out_type=jax.ShapeDtypeStruct((T, H), jnp.float32),
        mesh=mesh,
        scratch_types=[
            pltpu.VMEM((CHUNK,), jnp.int32),
            pltpu.VMEM((CHUNK,), jnp.float32),
            pltpu.VMEM((2, 16, H), jnp.float32),
            pltpu.VMEM((2, 8, H), jnp.float32),
            pltpu.SemaphoreType.DMA,
            pltpu.SemaphoreType.DMA,
            pltpu.SemaphoreType.DMA,
            pltpu.SemaphoreType.DMA,
        ],
        compiler_params=pltpu.CompilerParams(needs_layout_passes=False),
        name="moe_combine",
    )(ygflat, pos, topv_flat)


def kernel(hidden_states, gate_w, w1, w2, w3):
    B, S, Hd = hidden_states.shape
    x = hidden_states.reshape(-1, Hd)
    logits, topi, topv = _router(x, gate_w)
    xg, pos, bexp = _dispatch(topi.reshape(-1), x)
    yg = _mlp(bexp, xg, w1, w3, w2)
    out = _combine(yg, pos, topv.reshape(-1))
    return out.reshape(B, S, Hd), logits


# speedup vs baseline: 1.7524x; 1.0356x over previous
import functools

import jax
import jax.numpy as jnp
from jax import lax
from jax.experimental import pallas as pl
from jax.experimental.pallas import tpu as pltpu
from jax.experimental.pallas import tpu_sc as plsc

H = 2048
FF = 4096
E = 8
TOPK = 2
T = 4096
A = T * TOPK
BLK = 256
G = A + E * BLK
NBLK = G // BLK
NBPAD = 48
FFT = 1024
NF = FF // FFT

NC = 2
NS = 16
NW = NC * NS
CHUNK = A // NW


def _router_body(x_ref, gw_ref, logits_ref, topi_ref, topv_ref):
    x = x_ref[...]
    logits = lax.dot_general(x, gw_ref[...], (((1,), (1,)), ((), ())),
                             preferred_element_type=jnp.float32)
    logits_ref[...] = logits
    bt = logits.shape[0]
    idx = lax.broadcasted_iota(jnp.int32, (bt, E), 1)
    m1 = jnp.max(logits, axis=1, keepdims=True)
    i1 = jnp.min(jnp.where(logits == m1, idx, E), axis=1, keepdims=True)
    l2 = jnp.where(idx == i1, -jnp.inf, logits)
    m2 = jnp.max(l2, axis=1, keepdims=True)
    i2 = jnp.min(jnp.where(l2 == m2, idx, E), axis=1, keepdims=True)
    s = 1.0 / (1.0 + jnp.exp(m2 - m1))
    topi_ref[...] = jnp.concatenate([i1, i2], axis=1)
    topv_ref[...] = jnp.concatenate([s, 1.0 - s], axis=1)


def _router(x, gate_w):
    bt = T // 4
    return pl.pallas_call(
        _router_body,
        grid=(4,),
        in_specs=[
            pl.BlockSpec((bt, H), lambda i: (i, 0)),
            pl.BlockSpec((E, H), lambda i: (0, 0)),
        ],
        out_specs=[
            pl.BlockSpec((bt, E), lambda i: (i, 0)),
            pl.BlockSpec((bt, TOPK), lambda i: (i, 0)),
            pl.BlockSpec((bt, TOPK), lambda i: (i, 0)),
        ],
        out_shape=[
            jax.ShapeDtypeStruct((T, E), jnp.float32),
            jax.ShapeDtypeStruct((T, TOPK), jnp.int32),
            jax.ShapeDtypeStruct((T, TOPK), jnp.float32),
        ],
        name="moe_router",
    )(x, gate_w)


def _dispatch_body(topi_hbm, x_hbm, xg_hbm, pos_hbm, bexp_hbm,
                   ids_v, pos_v, rows_v, bexp_v, *sems):
    wid = lax.axis_index("s") * NC + lax.axis_index("c")
    i32 = jnp.int32
    one16 = jnp.ones((16,), i32)
    zero16 = jnp.zeros((16,), i32)
    pltpu.sync_copy(topi_hbm, ids_v)

    def count_body(i, accs):
        v = ids_v[pl.ds(i * 16, 16)]
        return tuple(accs[e] + jnp.where(v == e, one16, zero16)
                     for e in range(E))

    zeros8 = tuple(jnp.zeros((16,), i32) for _ in range(E))
    acc_tot = lax.fori_loop(0, A // 16, count_body, zeros8)
    acc_pre = lax.fori_loop(0, wid * (CHUNK // 16), count_body, zeros8)
    counts = [jnp.sum(acc_tot[e]) for e in range(E)]
    prefix = [jnp.sum(acc_pre[e]) for e in range(E)]

    offs = []
    ends = []
    off = jnp.int32(0)
    for e in range(E):
        offs.append(off)
        off = off + ((counts[e] + (BLK - 1)) // BLK) * BLK
        ends.append(off)

    iota16 = lax.iota(i32, 16)
    base = [offs[e] + prefix[e] for e in range(E)]
    nch = CHUNK // 16
    gh = {}
    sh = {}
    tok0 = (wid * CHUNK + iota16) // TOPK
    gh[0] = pltpu.async_copy(x_hbm.at[tok0], rows_v.at[0], sems[0])
    for v in range(nch):
        buf = v % 2
        j0 = wid * CHUNK + v * 16
        if v + 1 < nch:
            if v >= 1:
                sh[v - 1].wait()
            tokn = (j0 + 16 + iota16) // TOPK
            gh[v + 1] = pltpu.async_copy(
                x_hbm.at[tokn], rows_v.at[1 - buf], sems[1 - buf])
        ids16 = ids_v[pl.ds(j0, 16)]
        pos16 = jnp.zeros((16,), i32)
        for e in range(E):
            m = ids16 == e
            mi = jnp.where(m, one16, zero16)
            incl = plsc.cumsum(mi)
            pos16 = jnp.where(m, base[e] + (incl - mi), pos16)
            base[e] = base[e] + jnp.sum(mi)
        pos_v[pl.ds(v * 16, 16)] = pos16
        gh[v].wait()
        sh[v] = pltpu.async_copy(rows_v.at[buf], xg_hbm.at[pos16], sems[2 + buf])
    sh[nch - 2].wait()
    sh[nch - 1].wait()

    pltpu.sync_copy(pos_v, pos_hbm.at[pl.ds(pl.multiple_of(wid * CHUNK, 8), CHUNK)])

    @pl.when(wid == 0)
    def _():
        for i in range(NBPAD // 16):
            rowstart = (iota16 + i * 16) * BLK
            acc = jnp.zeros((16,), i32)
            for e in range(E):
                acc = acc + jnp.where(rowstart >= ends[e], one16, zero16)
            bexp_v[pl.ds(i * 16, 16)] = acc
        pltpu.sync_copy(bexp_v, bexp_hbm)


def _dispatch(topi_flat, x):
    mesh = plsc.VectorSubcoreMesh(core_axis_name="c", subcore_axis_name="s")
    return pl.kernel(
        _dispatch_body,
        out_type=(
            jax.ShapeDtypeStruct((G, H), jnp.float32),
            jax.ShapeDtypeStruct((A,), jnp.int32),
            jax.ShapeDtypeStruct((NBPAD,), jnp.int32),
        ),
        mesh=mesh,
        scratch_types=[
            pltpu.VMEM((A,), jnp.int32),
            pltpu.VMEM((CHUNK,), jnp.int32),
            pltpu.VMEM((2, 16, H), jnp.float32),
            pltpu.VMEM((NBPAD,), jnp.int32),
            pltpu.SemaphoreType.DMA,
            pltpu.SemaphoreType.DMA,
            pltpu.SemaphoreType.DMA,
            pltpu.SemaphoreType.DMA,
        ],
        compiler_params=pltpu.CompilerParams(needs_layout_passes=False),
        name="moe_dispatch",
    )(topi_flat, x)


def _ffn_part(x, w1_ref, w3_ref, w2_ref):
    a = lax.dot_general(x, w1_ref[0], (((1,), (1,)), ((), ())),
                        preferred_element_type=jnp.float32)
    c = lax.dot_general(x, w3_ref[0], (((1,), (1,)), ((), ())),
                        preferred_element_type=jnp.float32)
    h = (a * jax.nn.sigmoid(a)) * c
    return lax.dot_general(h, w2_ref[0], (((1,), (1,)), ((), ())),
                           preferred_element_type=jnp.float32)


def _mlp_first_body(be_ref, xg_ref, w1_ref, w3_ref, w2_ref, xbf_ref, ybf_ref):
    b = pl.program_id(0)
    e = be_ref[b]

    @pl.when(e < E)
    def _():
        x = xg_ref[...]
        xbf_ref[...] = x.astype(jnp.bfloat16)
        ybf_ref[...] = _ffn_part(x, w1_ref, w3_ref, w2_ref).astype(jnp.bfloat16)


def _mlp_mid_body(be_ref, xbf_ref, w1_ref, w3_ref, w2_ref, yin_ref, ybf_ref):
    b = pl.program_id(0)
    e = be_ref[b]

    @pl.when(e < E)
    def _():
        x = xbf_ref[...].astype(jnp.float32)
        y = yin_ref[...].astype(jnp.float32) + _ffn_part(x, w1_ref, w3_ref, w2_ref)
        ybf_ref[...] = y.astype(jnp.bfloat16)


def _mlp_last_body(be_ref, xbf_ref, w1_ref, w3_ref, w2_ref, yin_ref, yg_ref):
    b = pl.program_id(0)
    e = be_ref[b]

    @pl.when(e < E)
    def _():
        x = xbf_ref[...].astype(jnp.float32)
        yg_ref[...] = (yin_ref[...].astype(jnp.float32)
                       + _ffn_part(x, w1_ref, w3_ref, w2_ref))


def _mlp(bexp, xg, w1, w3, w2):
    def specs(f):
        return [
            pl.BlockSpec((1, FFT, H),
                         lambda b, be, f=f: (jnp.minimum(be[b], E - 1), f, 0)),
            pl.BlockSpec((1, FFT, H),
                         lambda b, be, f=f: (jnp.minimum(be[b], E - 1), f, 0)),
            pl.BlockSpec((1, H, FFT),
                         lambda b, be, f=f: (jnp.minimum(be[b], E - 1), 0, f)),
        ]

    read_spec = pl.BlockSpec((BLK, H),
                             lambda b, be: (jnp.where(be[b] < E, b, 0), 0))
    write_spec = pl.BlockSpec((BLK, H), lambda b, be: (b, 0))
    cp = pltpu.CompilerParams(dimension_semantics=("arbitrary",),
                              vmem_limit_bytes=63 * 1024 * 1024)

    grid_spec = pltpu.PrefetchScalarGridSpec(
        num_scalar_prefetch=1, grid=(NBLK,),
        in_specs=[read_spec] + specs(0),
        out_specs=[write_spec, write_spec])
    xbf, ybf = pl.pallas_call(
        _mlp_first_body,
        grid_spec=grid_spec,
        out_shape=[jax.ShapeDtypeStruct((G, H), jnp.bfloat16),
                   jax.ShapeDtypeStruct((G, H), jnp.bfloat16)],
        compiler_params=cp,
        name="moe_mlp_f0",
    )(bexp, xg, w1, w3, w2)

    for f in range(1, NF - 1):
        grid_spec = pltpu.PrefetchScalarGridSpec(
            num_scalar_prefetch=1, grid=(NBLK,),
            in_specs=[read_spec] + specs(f) + [write_spec],
            out_specs=write_spec)
        ybf = pl.pallas_call(
            _mlp_mid_body,
            grid_spec=grid_spec,
            out_shape=jax.ShapeDtypeStruct((G, H), jnp.bfloat16),
            input_output_aliases={5: 0},
            compiler_params=cp,
            name=f"moe_mlp_f{f}",
        )(bexp, xbf, w1, w3, w2, ybf)

    grid_spec = pltpu.PrefetchScalarGridSpec(
        num_scalar_prefetch=1, grid=(NBLK,),
        in_specs=[read_spec] + specs(NF - 1) + [write_spec],
        out_specs=write_spec)
    return pl.pallas_call(
        _mlp_last_body,
        grid_spec=grid_spec,
        out_shape=jax.ShapeDtypeStruct((G, H), jnp.float32),
        compiler_params=cp,
        name=f"moe_mlp_f{NF - 1}",
    )(bexp, xbf, w1, w3, w2, ybf)


def _combine_body(yg_hbm, pos_hbm, topv_hbm, out_hbm,
                  posc, tvc, rows_v, out_v, *sems):
    wid = lax.axis_index("s") * NC + lax.axis_index("c")
    nch = CHUNK // 16
    base8 = pl.multiple_of(wid * CHUNK, 8)
    pltpu.sync_copy(pos_hbm.at[pl.ds(base8, CHUNK)], posc)
    pltpu.sync_copy(topv_hbm.at[pl.ds(base8, CHUNK)], tvc)
    gh = {}
    so = {}
    gh[0] = pltpu.async_copy(yg_hbm.at[posc[pl.ds(0, 16)]], rows_v.at[0],
                             sems[0])
    for c in range(nch):
        buf = c % 2
        if c + 1 < nch:
            pvn = posc[pl.ds((c + 1) * 16, 16)]
            gh[c + 1] = pltpu.async_copy(yg_hbm.at[pvn], rows_v.at[1 - buf],
                                         sems[1 - buf])
        if c >= 2:
            so[c - 2].wait()
        gh[c].wait()
        tv = tvc[pl.ds(c * 16, 16)]
        svs = [tv[k] for k in range(16)]

        def blend(j, _, buf=buf, svs=svs):
            for i in range(8):
                a = rows_v[buf, 2 * i, pl.ds(j * 16, 16)]
                b = rows_v[buf, 2 * i + 1, pl.ds(j * 16, 16)]
                out_v[buf, i, pl.ds(j * 16, 16)] = (
                    svs[2 * i] * a + svs[2 * i + 1] * b)
            return 0

        lax.fori_loop(0, H // 16, blend, 0)
        so[c] = pltpu.async_copy(
            out_v.at[buf],
            out_hbm.at[pl.ds(pl.multiple_of(base8 // TOPK + c * 8, 8), 8)],
            sems[2 + buf])
    so[nch - 2].wait()
    so[nch - 1].wait()


def _combine(ygflat, pos, topv_flat):
    mesh = plsc.VectorSubcoreMesh(core_axis_name="c", subcore_axis_name="s")
    return pl.kernel(
        _combine_body,
        out_type=jax.ShapeDtypeStruct((T, H), jnp.float32),
        mesh=mesh,
        scratch_types=[
            pltpu.VMEM((CHUNK,), jnp.int32),
            pltpu.VMEM((CHUNK,), jnp.float32),
            pltpu.VMEM((2, 16, H), jnp.float32),
            pltpu.VMEM((2, 8, H), jnp.float32),
            pltpu.SemaphoreType.DMA,
            pltpu.SemaphoreType.DMA,
            pltpu.SemaphoreType.DMA,
            pltpu.SemaphoreType.DMA,
        ],
        compiler_params=pltpu.CompilerParams(needs_layout_passes=False),
        name="moe_combine",
    )(ygflat, pos, topv_flat)


def kernel(hidden_states, gate_w, w1, w2, w3):
    B, S, Hd = hidden_states.shape
    x = hidden_states.reshape(-1, Hd)
    logits, topi, topv = _router(x, gate_w)
    xg, pos, bexp = _dispatch(topi.reshape(-1), x)
    yg = _mlp(bexp, xg, w1, w3, w2)
    out = _combine(yg, pos, topv.reshape(-1))
    return out.reshape(B, S, Hd), logits
